# baseline (speedup 1.0000x reference)
import functools

import jax
import jax.numpy as jnp
from jax import lax
from jax.experimental import pallas as pl
from jax.experimental.pallas import tpu as pltpu
from jax.experimental.pallas import tpu_sc as plsc


def _sc_body(idx_hbm, rig_hbm, cam_hbm, proj_hbm, pose_out, proj_out,
             idx_v, rig_v, cam_v, proj_v, pose_v):
    tile0 = (lax.axis_index("c") == 0) & (lax.axis_index("s") == 0)

    @pl.when(tile0)
    def _():
        pltpu.sync_copy(idx_hbm, idx_v.at[pl.ds(0, 2)])
        idx_vec = idx_v[...]
        frame = idx_vec[0]
        cam = idx_vec[1]
        pltpu.sync_copy(rig_hbm.at[frame], rig_v)
        pltpu.sync_copy(cam_hbm.at[cam], cam_v)
        pltpu.sync_copy(proj_hbm.at[cam], proj_v)

        lane = lax.iota(jnp.int32, 16)
        row4 = lane & 12
        col4 = lane & 3
        acc = plsc.load_gather(cam_v, [row4]) * plsc.load_gather(rig_v, [col4])
        for k in range(1, 4):
            a_k = plsc.load_gather(cam_v, [row4 + k])
            b_k = plsc.load_gather(rig_v, [4 * k + col4])
            acc = acc + a_k * b_k
        pose_v[...] = acc

        pltpu.sync_copy(pose_v, pose_out)
        pltpu.sync_copy(proj_v, proj_out)


@jax.jit
def _sc_call(image_idx, rig, cam, proj):
    mesh = plsc.VectorSubcoreMesh(core_axis_name="c", subcore_axis_name="s")
    return pl.kernel(
        _sc_body,
        mesh=mesh,
        out_type=(
            jax.ShapeDtypeStruct((16,), jnp.float32),
            jax.ShapeDtypeStruct((9,), jnp.float32),
        ),
        scratch_types=[
            pltpu.VMEM((16,), jnp.int32),
            pltpu.VMEM((16,), jnp.float32),
            pltpu.VMEM((16,), jnp.float32),
            pltpu.VMEM((9,), jnp.float32),
            pltpu.VMEM((16,), jnp.float32),
        ],
        compiler_params=pltpu.CompilerParams(needs_layout_passes=False),
    )(image_idx, rig, cam, proj)


def kernel(image_idx, rig_t_world, camera_t_rig, projection):
    f = rig_t_world.shape[0]
    c = camera_t_rig.shape[0]
    pose_flat, proj_flat = _sc_call(
        image_idx.astype(jnp.int32),
        rig_t_world.reshape(f, 16),
        camera_t_rig.reshape(c, 16),
        projection.reshape(c, 9),
    )
    return pose_flat.reshape(1, 4, 4), proj_flat.reshape(3, 3)


# speedup vs baseline: 1.0530x; 1.0530x over previous
import functools

import jax
import jax.numpy as jnp
from jax import lax
from jax.experimental import pallas as pl
from jax.experimental.pallas import tpu as pltpu
from jax.experimental.pallas import tpu_sc as plsc


def _sc_body(idx_hbm, rig_hbm, cam_hbm, proj_hbm, pose_out, proj_out,
             idx_v, rig_v, cam_v, proj_v, pose_v, sem):
    pltpu.sync_copy(idx_hbm, idx_v.at[pl.ds(0, 2)])
    idx_vec = idx_v[...]
    frame = idx_vec[0]
    cam = idx_vec[1]
    c1 = pltpu.async_copy(rig_hbm.at[frame], rig_v, sem)
    c2 = pltpu.async_copy(cam_hbm.at[cam], cam_v, sem)
    c3 = pltpu.async_copy(proj_hbm.at[cam], proj_v, sem)
    c1.wait()
    c2.wait()
    c3.wait()

    lane = lax.iota(jnp.int32, 16)
    row4 = lane & 12
    col4 = lane & 3
    acc = plsc.load_gather(cam_v, [row4]) * plsc.load_gather(rig_v, [col4])
    for k in range(1, 4):
        a_k = plsc.load_gather(cam_v, [row4 + k])
        b_k = plsc.load_gather(rig_v, [4 * k + col4])
        acc = acc + a_k * b_k
    pose_v[...] = acc

    c4 = pltpu.async_copy(pose_v, pose_out, sem)
    c5 = pltpu.async_copy(proj_v, proj_out, sem)
    c4.wait()
    c5.wait()


@jax.jit
def _sc_call(image_idx, rig, cam, proj):
    mesh = plsc.VectorSubcoreMesh(core_axis_name="c", subcore_axis_name="s",
                                  num_cores=1, num_subcores=1)
    return pl.kernel(
        _sc_body,
        mesh=mesh,
        out_type=(
            jax.ShapeDtypeStruct((16,), jnp.float32),
            jax.ShapeDtypeStruct((9,), jnp.float32),
        ),
        scratch_types=[
            pltpu.VMEM((16,), jnp.int32),
            pltpu.VMEM((16,), jnp.float32),
            pltpu.VMEM((16,), jnp.float32),
            pltpu.VMEM((9,), jnp.float32),
            pltpu.VMEM((16,), jnp.float32),
            pltpu.SemaphoreType.DMA,
        ],
        compiler_params=pltpu.CompilerParams(needs_layout_passes=False),
    )(image_idx, rig, cam, proj)


def kernel(image_idx, rig_t_world, camera_t_rig, projection):
    f = rig_t_world.shape[0]
    c = camera_t_rig.shape[0]
    pose_flat, proj_flat = _sc_call(
        image_idx.astype(jnp.int32),
        rig_t_world.reshape(f, 16),
        camera_t_rig.reshape(c, 16),
        projection.reshape(c, 9),
    )
    return pose_flat.reshape(1, 4, 4), proj_flat.reshape(3, 3)
